# baseline (speedup 1.0000x reference)
import functools

import jax
import jax.numpy as jnp
from jax import lax
from jax.experimental import pallas as pl
from jax.experimental.pallas import tpu as pltpu
from jax.experimental.pallas import tpu_sc as plsc

B = 16384
F = 512
D = 32
NC = 2
NS = 16
NW = NC * NS
BPW = B // NW

BLK = 1024


def _sc_gather_body(uidx, pidx, nidx, uemb, iemb, ue_out, pe_out, ne_out,
                    idx_v, rows_v, sem):
    wid = lax.axis_index("s") * NC + lax.axis_index("c")
    base = wid * BPW

    pltpu.sync_copy(uidx.at[pl.ds(base, BPW)], idx_v)
    pltpu.async_copy(uemb.at[idx_v], rows_v, sem).wait()
    pltpu.sync_copy(rows_v, ue_out.at[pl.ds(base, BPW)])

    pltpu.sync_copy(pidx.at[pl.ds(base, BPW)], idx_v)
    pltpu.async_copy(iemb.at[idx_v], rows_v, sem).wait()
    pltpu.sync_copy(rows_v, pe_out.at[pl.ds(base, BPW)])

    pltpu.sync_copy(nidx.at[pl.ds(base, BPW)], idx_v)
    pltpu.async_copy(iemb.at[idx_v], rows_v, sem).wait()
    pltpu.sync_copy(rows_v, ne_out.at[pl.ds(base, BPW)])


def _sc_gather(uidx, pidx, nidx, uemb, iemb):
    mesh = plsc.VectorSubcoreMesh(core_axis_name="c", subcore_axis_name="s")
    out = jax.ShapeDtypeStruct((B, D), jnp.float32)
    fn = functools.partial(
        pl.kernel,
        mesh=mesh,
        out_type=(out, out, out),
        scratch_types=[
            pltpu.VMEM((BPW,), jnp.int32),
            pltpu.VMEM((BPW, D), jnp.float32),
            pltpu.SemaphoreType.DMA,
        ],
        compiler_params=pltpu.CompilerParams(use_tc_tiling_on_sc=False),
    )(_sc_gather_body)
    return fn(uidx, pidx, nidx, uemb, iemb)


def _tc_body(uf, pf, nf, ue, pe, ne, umap, imap, w1, b1, w2, out):
    i = pl.program_id(0)
    un = (uf[...] - 2.5) * 0.4
    pn = (pf[...] - 2.5) * 0.4
    nn = (nf[...] - 2.5) * 0.4
    um = jnp.dot(un, umap[...], preferred_element_type=jnp.float32)
    pm = jnp.dot(pn, imap[...], preferred_element_type=jnp.float32)
    nm = jnp.dot(nn, imap[...], preferred_element_type=jnp.float32)

    uip = ue[...] * pe[...]
    uin = ue[...] * ne[...]
    fp = um * pm
    fn_ = um * nm

    w1v = w1[...]
    w1a = w1v[:, :D]
    w1b = w1v[:, D:]
    b1v = b1[...]
    w2v = w2[...]

    cdims = (((1,), (1,)), ((), ()))
    hp = lax.dot_general(uip, w1a, cdims,
                         preferred_element_type=jnp.float32)
    hp = hp + lax.dot_general(fp, w1b, cdims,
                              preferred_element_type=jnp.float32)
    hp = jnp.maximum(hp + b1v, 0.0)
    hn = lax.dot_general(uin, w1a, cdims,
                         preferred_element_type=jnp.float32)
    hn = hn + lax.dot_general(fn_, w1b, cdims,
                              preferred_element_type=jnp.float32)
    hn = jnp.maximum(hn + b1v, 0.0)

    d = lax.dot_general(hp - hn, w2v, cdims,
                        preferred_element_type=jnp.float32)
    part = jnp.sum(jnp.minimum(d, 0.0) - jnp.log1p(jnp.exp(-jnp.abs(d))))

    @pl.when(i == 0)
    def _():
        out[0, 0] = 0.0

    out[0, 0] += part


def _tc_main(uf, pf, nf, ue, pe, ne, umap, imap, w1, b1, w2):
    grid = B // BLK
    feat_spec = pl.BlockSpec((BLK, F), lambda i: (i, 0))
    emb_spec = pl.BlockSpec((BLK, D), lambda i: (i, 0))
    def full(shape):
        return pl.BlockSpec(shape, lambda i: tuple(0 for _ in shape))
    total = pl.pallas_call(
        _tc_body,
        grid=(grid,),
        in_specs=[feat_spec, feat_spec, feat_spec,
                  emb_spec, emb_spec, emb_spec,
                  full((F, D)), full((F, D)), full((10, 64)),
                  full((1, 10)), full((1, 10))],
        out_specs=pl.BlockSpec((1, 1), lambda i: (0, 0),
                               memory_space=pltpu.SMEM),
        out_shape=jax.ShapeDtypeStruct((1, 1), jnp.float32),
    )(uf, pf, nf, ue, pe, ne, umap, imap, w1, b1, w2)
    return total


def kernel(user_batch, user_feature_batch, pos_item_batch,
           pos_item_feature_batch, neg_item_batch, neg_item_feature_batch,
           user_emb, item_emb, user_map, item_map,
           fc1_w, fc1_b, fc2_w, fc2_b):
    uidx = user_batch.astype(jnp.int32)
    pidx = pos_item_batch.astype(jnp.int32)
    nidx = neg_item_batch.astype(jnp.int32)

    ue, pe, ne = _sc_gather(uidx, pidx, nidx, user_emb, item_emb)

    total = _tc_main(user_feature_batch, pos_item_feature_batch,
                     neg_item_feature_batch, ue, pe, ne,
                     user_map, item_map, fc1_w,
                     fc1_b.reshape(1, 10), fc2_w)
    return -total[0, 0] / B
